# baseline (speedup 1.0000x reference)
import functools

import jax
import jax.numpy as jnp
from jax import lax
from jax.experimental import pallas as pl
from jax.experimental.pallas import tpu as pltpu
from jax.experimental.pallas import tpu_sc as plsc

VOCAB = 1_000_000
HIDDEN = 64
NI = 4096
NJ = 200
B_TOTAL = NI * NJ

_info = plsc.get_sparse_core_info()
_NC, _NS = _info.num_cores, _info.num_subcores
_NW = _NC * _NS
_B_PER_W = B_TOTAL // _NW
_C = 512
_BLOCKS = _C // 128
_NBUF = 2
_N_CH = _B_PER_W // _C


def _gather_body(table_hbm, xt_hbm, out_hbm, idx_v, rows_v, tbuf, gsems, osem):
    wid = lax.axis_index("s") * _NC + lax.axis_index("c")
    base_t = wid * _B_PER_W
    tbl = table_hbm
    iota = lax.iota(jnp.int32, 16)

    def fetch(g, b):
        t0 = base_t + g * _C
        pltpu.sync_copy(xt_hbm.at[pl.ds(t0, _C)], idx_v.at[b])
        pltpu.async_copy(tbl.at[idx_v.at[b]], rows_v.at[b], gsems[b])

    def process(g, b):
        t0 = base_t + g * _C
        j = t0 // NI
        tc0 = (t0 % NI) // 128

        pltpu.make_async_copy(tbl.at[idx_v.at[b]], rows_v.at[b], gsems[b]).wait()

        @pl.when(g > 0)
        def _():
            for blk in range(_BLOCKS):
                pltpu.make_async_copy(
                    tbuf.at[pl.ds(blk * 8, 8), :],
                    out_hbm.at[j, :, tc0 + blk, :],
                    osem,
                ).wait()

        @plsc.parallel_loop(0, HIDDEN, unroll=8)
        def _(h):
            hv = jnp.broadcast_to(h, (16,))
            tr = h // 8
            col0 = (h % 8) * 128
            for blk in range(_BLOCKS):
                for ic in range(8):
                    ivec = iota + (blk * 128 + ic * 16)
                    v = plsc.load_gather(rows_v.at[b], [ivec, hv])
                    tbuf[blk * 8 + tr, pl.ds(col0 + ic * 16, 16)] = v

        for blk in range(_BLOCKS):
            pltpu.async_copy(
                tbuf.at[pl.ds(blk * 8, 8), :],
                out_hbm.at[j, :, tc0 + blk, :],
                osem,
            )

    for b in range(_NBUF):
        fetch(b, b)

    @pl.loop(0, _N_CH - _NBUF, step=_NBUF)
    def _(g):
        for b in range(_NBUF):
            process(g + b, b)
            fetch(g + b + _NBUF, b)

    for b in range(_NBUF):
        g = _N_CH - _NBUF + b
        process(g, b)

    g = _N_CH - 1
    t0 = base_t + g * _C
    j = t0 // NI
    tc0 = (t0 % NI) // 128
    for blk in range(_BLOCKS):
        pltpu.make_async_copy(
            tbuf.at[pl.ds(blk * 8, 8), :],
            out_hbm.at[j, :, tc0 + blk, :],
            osem,
        ).wait()


@jax.jit
def _gather(table_flat, xt_flat):
    k = functools.partial(
        pl.kernel,
        out_type=jax.ShapeDtypeStruct((NJ, 8, 32, 1024), jnp.float32),
        mesh=plsc.VectorSubcoreMesh(core_axis_name="c", subcore_axis_name="s"),
        scratch_types=[
            pltpu.VMEM((_NBUF, _C), jnp.int32),
            pltpu.VMEM((_NBUF, _C, HIDDEN), jnp.float32),
            pltpu.VMEM((_BLOCKS * 8, 1024), jnp.float32),
            [pltpu.SemaphoreType.DMA] * _NBUF,
            pltpu.SemaphoreType.DMA,
        ],
        compiler_params=pltpu.CompilerParams(
            use_tc_tiling_on_sc=False, needs_layout_passes=False
        ),
    )(_gather_body)
    return k(table_flat, xt_flat)


def kernel(x, embedding_weight):
    xt_flat = x.T.reshape(-1).astype(jnp.int32)
    raw = _gather(embedding_weight, xt_flat)
    raw5 = raw.reshape(NJ, 8, 32, 8, 128)
    return raw5.transpose(2, 4, 0, 1, 3).reshape(NI, NJ, HIDDEN)


# speedup vs baseline: 1.3483x; 1.3483x over previous
import functools

import jax
import jax.numpy as jnp
from jax import lax
from jax.experimental import pallas as pl
from jax.experimental.pallas import tpu as pltpu
from jax.experimental.pallas import tpu_sc as plsc

VOCAB = 1_000_000
HIDDEN = 64
NI = 4096
NJ = 200
B_TOTAL = NI * NJ

_info = plsc.get_sparse_core_info()
_NC, _NS = _info.num_cores, _info.num_subcores
_NW = _NC * _NS
_B_PER_W = B_TOTAL // _NW
_C = 512
_BLOCKS = _C // 128
_NBUF = 2
_N_CH = _B_PER_W // _C


def _gather_body(table_hbm, xt_hbm, out_hbm, idx_v, rows_v, tbuf, gsems, osem):
    wid = lax.axis_index("s") * _NC + lax.axis_index("c")
    base_t = wid * _B_PER_W
    tbl = table_hbm
    iota = lax.iota(jnp.int32, 16)
    perm_c = [lax.rem(iota + d, 16) for d in range(16)]
    rot_c = [lax.rem(iota + (16 - d), 16) for d in range(16)]
    scol_c = [lax.rem(iota, 8) * 128 + rot_c[d] for d in range(16)]
    srow_c = iota // 8

    def fetch(g, b):
        t0 = base_t + g * _C
        pltpu.sync_copy(xt_hbm.at[pl.ds(t0, _C)], idx_v.at[b])
        pltpu.async_copy(tbl.at[idx_v.at[b]], rows_v.at[b], gsems[b])

    def process(g, b):
        t0 = base_t + g * _C
        j = t0 // NI
        tc0 = (t0 % NI) // 128

        pltpu.make_async_copy(tbl.at[idx_v.at[b]], rows_v.at[b], gsems[b]).wait()

        @pl.when(g > 0)
        def _():
            for blk in range(_BLOCKS):
                pltpu.make_async_copy(
                    tbuf.at[pl.ds(blk * 8, 8), :],
                    out_hbm.at[j, :, tc0 + blk, :],
                    osem,
                ).wait()

        @plsc.parallel_loop(0, _C // 16, unroll=1)
        def _(tt):
            tvec = iota + tt * 16
            blk = tt // 8
            c0 = (tt % 8) * 16
            for hh in range(HIDDEN // 16):
                srowv = srow_c + (blk * 8 + 2 * hh)
                for d in range(16):
                    hv = perm_c[d] + hh * 16
                    v = plsc.load_gather(rows_v.at[b], [tvec, hv])
                    u = jnp.take_along_axis(v, rot_c[d], axis=0)
                    scol = scol_c[d] + c0
                    plsc.store_scatter(tbuf, [srowv, scol], u)

        for blk in range(_BLOCKS):
            pltpu.async_copy(
                tbuf.at[pl.ds(blk * 8, 8), :],
                out_hbm.at[j, :, tc0 + blk, :],
                osem,
            )

    for b in range(_NBUF):
        fetch(b, b)

    @pl.loop(0, _N_CH, step=_NBUF)
    def _(g):
        for b in range(_NBUF):
            process(g + b, b)
            nxt = g + b + _NBUF

            @pl.when(nxt < _N_CH)
            def _():
                fetch(nxt, b)

    g = _N_CH - 1
    t0 = base_t + g * _C
    j = t0 // NI
    tc0 = (t0 % NI) // 128
    for blk in range(_BLOCKS):
        pltpu.make_async_copy(
            tbuf.at[pl.ds(blk * 8, 8), :],
            out_hbm.at[j, :, tc0 + blk, :],
            osem,
        ).wait()


@jax.jit
def _gather(table_flat, xt_flat):
    k = functools.partial(
        pl.kernel,
        out_type=jax.ShapeDtypeStruct((NJ, 8, 32, 1024), jnp.float32),
        mesh=plsc.VectorSubcoreMesh(core_axis_name="c", subcore_axis_name="s"),
        scratch_types=[
            pltpu.VMEM((_NBUF, _C), jnp.int32),
            pltpu.VMEM((_NBUF, _C, HIDDEN), jnp.float32),
            pltpu.VMEM((_BLOCKS * 8, 1024), jnp.float32),
            [pltpu.SemaphoreType.DMA] * _NBUF,
            pltpu.SemaphoreType.DMA,
        ],
        compiler_params=pltpu.CompilerParams(
            use_tc_tiling_on_sc=False, needs_layout_passes=False
        ),
    )(_gather_body)
    return k(table_flat, xt_flat)


_BK = 512


def _detile_body(x_ref, o_ref):
    o_ref[...] = x_ref[...].T.reshape(_BK // 2, 128)


@jax.jit
def _detile(t64):
    return pl.pallas_call(
        _detile_body,
        grid=(VOCAB // _BK,),
        in_specs=[pl.BlockSpec((HIDDEN, _BK), lambda i: (0, i))],
        out_specs=pl.BlockSpec((_BK // 2, 128), lambda i: (i, 0)),
        out_shape=jax.ShapeDtypeStruct((VOCAB // 2, 128), jnp.float32),
    )(t64)


def kernel(x, embedding_weight):
    xt_flat = x.T.reshape(-1).astype(jnp.int32)
    raw = _gather(embedding_weight, xt_flat)
    raw5 = raw.reshape(NJ, 8, 32, 8, 128)
    return raw5.transpose(2, 4, 0, 1, 3).reshape(NI, NJ, HIDDEN)
